# Initial kernel scaffold; baseline (speedup 1.0000x reference)
import jax
import jax.numpy as jnp
from jax.experimental import pallas as pl


def kernel(xij, r, attn, edge_index):
    raise NotImplementedError("write your pallas kernel here")


# speedup vs baseline: 3.4928x; 3.4928x over previous
import functools

import jax
import jax.numpy as jnp
from jax import lax
from jax.experimental import pallas as pl
from jax.experimental.pallas import tpu as pltpu
from jax.experimental.pallas import tpu_sc as plsc

NC = 2
NS = 16
L = 16
CHUNK = 128

f32 = jnp.float32
i32 = jnp.int32


def _splat(ref, i):
    return plsc.load_gather(ref, [jnp.full((L,), i, i32)])


def _rn_tc(rT):
    def body(r_ref, o_ref):
        x = r_ref[...]
        n2 = x[0:1, :] * x[0:1, :] + x[1:2, :] * x[1:2, :] + x[2:3, :] * x[2:3, :]
        o_ref[...] = x * lax.rsqrt(n2 + 1e-35)

    return pl.pallas_call(
        body, out_shape=jax.ShapeDtypeStruct(rT.shape, f32))(rT)


def _dinv_tc(d0, d1):
    def body(a_ref, b_ref, o_ref):
        o_ref[...] = 1.0 / (a_ref[...] + b_ref[...] + 1e-9)

    return pl.pallas_call(
        body, out_shape=jax.ShapeDtypeStruct(d0.shape, f32))(d0, d1)


@functools.lru_cache(maxsize=None)
def _build_k2(NPAD, EPAD, C):
    EW = EPAD // (NC * NS)
    NCHUNK = EW // CHUNK
    SLICE = NPAD // NS
    mesh = plsc.VectorSubcoreMesh(core_axis_name="c", subcore_axis_name="s")

    def body(src_hbm, dst_hbm, xij_hbm, rn16_hbm, attn_hbm,
             aexp_hbm, den_hbm,
             sidx, didx, xs, xd, rs16, rd16,
             xsum, rnS, rnD, aexp_v, attn_v, zbuf, den_sp, sem):
        cid = lax.axis_index("c")
        sid = lax.axis_index("s")
        wid = cid * NS + sid
        zeros16 = jnp.zeros((L,), f32)
        iota_s = lax.iota(i32, L) * CHUNK

        pltpu.sync_copy(attn_hbm, attn_v)

        def zbody(i, c):
            zbuf[pl.ds(i * L, L)] = zeros16
            return c
        lax.fori_loop(0, SLICE // L, zbody, 0)
        pltpu.sync_copy(zbuf, den_sp.at[pl.ds(sid * SLICE, SLICE)])
        plsc.subcore_barrier()

        def chunk_body(ch, c):
            off = wid * EW + ch * CHUNK
            pltpu.sync_copy(src_hbm.at[pl.ds(off, CHUNK)], sidx)
            pltpu.sync_copy(dst_hbm.at[pl.ds(off, CHUNK)], didx)
            cps = [pltpu.async_copy(xij_hbm.at[sidx], xs, sem),
                   pltpu.async_copy(xij_hbm.at[didx], xd, sem),
                   pltpu.async_copy(rn16_hbm.at[sidx], rs16, sem),
                   pltpu.async_copy(rn16_hbm.at[didx], rd16, sem)]
            for cp in cps:
                cp.wait()

            def e_body(e, cc):
                for j in range(C // L):
                    v = xs[e, pl.ds(j * L, L)] + xd[e, pl.ds(j * L, L)]
                    plsc.store_scatter(xsum, [iota_s + (j * L * CHUNK + e)], v)
                plsc.store_scatter(rnS, [iota_s + e], rs16[e, pl.ds(0, L)])
                plsc.store_scatter(rnD, [iota_s + e], rd16[e, pl.ds(0, L)])
                return cc
            lax.fori_loop(0, CHUNK, e_body, 0)

            def group(g, cc):
                g16 = g * L
                cos = (rnS[pl.ds(g16, L)] * rnD[pl.ds(g16, L)]
                       + rnS[pl.ds(CHUNK + g16, L)] * rnD[pl.ds(CHUNK + g16, L)]
                       + rnS[pl.ds(2 * CHUNK + g16, L)] * rnD[pl.ds(2 * CHUNK + g16, L)])
                c2x = cos + cos

                def kbody(k, carry):
                    t0, t1, a = carry
                    s = t1 + xsum[pl.ds(k * CHUNK + g16, L)]
                    a = a + _splat(attn_v, k) * (s / (1.0 + jnp.exp(-s)))
                    return (t1, c2x * t1 - t0, a)

                _, _, acc = lax.fori_loop(
                    0, C, kbody, (cos, jnp.ones((L,), f32),
                                  jnp.zeros((L,), f32)))
                aexp_v[pl.ds(g16, L)] = jnp.exp(acc)
                return cc
            lax.fori_loop(0, CHUNK // L, group, 0)

            pltpu.sync_copy(aexp_v, aexp_hbm.at[pl.ds(off, CHUNK)])
            pltpu.sync_copy(aexp_v, den_sp.at[didx], add=True)
            return c
        lax.fori_loop(0, NCHUNK, chunk_body, 0)

        plsc.subcore_barrier()
        pltpu.sync_copy(den_sp.at[pl.ds(sid * SLICE, SLICE)],
                        den_hbm.at[pl.ds(cid * NPAD + sid * SLICE, SLICE)])

    return pl.kernel(
        body,
        out_type=(jax.ShapeDtypeStruct((EPAD,), f32),
                  jax.ShapeDtypeStruct((NC * NPAD,), f32)),
        mesh=mesh,
        compiler_params=pltpu.CompilerParams(needs_layout_passes=False, use_tc_tiling_on_sc=False),
        scratch_types=[
            pltpu.VMEM((CHUNK,), i32),
            pltpu.VMEM((CHUNK,), i32),
            pltpu.VMEM((CHUNK, C), f32),
            pltpu.VMEM((CHUNK, C), f32),
            pltpu.VMEM((CHUNK, L), f32),
            pltpu.VMEM((CHUNK, L), f32),
            pltpu.VMEM((C * CHUNK,), f32),
            pltpu.VMEM((L * CHUNK,), f32),
            pltpu.VMEM((L * CHUNK,), f32),
            pltpu.VMEM((CHUNK,), f32),
            pltpu.VMEM((C,), f32),
            pltpu.VMEM((NPAD // NS,), f32),
            pltpu.VMEM_SHARED((NPAD,), f32),
            pltpu.SemaphoreType.DMA,
        ],
    )


@functools.lru_cache(maxsize=None)
def _build_k3(NPAD, EPAD, CH):
    EW3 = EPAD // NS
    NCHUNK3 = EW3 // CHUNK
    SLICE = NPAD // NS
    ZR = 64 if SLICE % 64 == 0 else 8
    NZ = SLICE // ZR
    mesh = plsc.VectorSubcoreMesh(core_axis_name="c", subcore_axis_name="s")

    def body(src2_hbm, dst_hbm, aexp_hbm, dinv16_hbm, xsplit_hbm,
             ft_hbm,
             sidx, didx, aexp_v, dv16, xsb, stage, zrow, ft_sp,
             sem):
        cid = lax.axis_index("c")
        sid = lax.axis_index("s")
        zeros16 = jnp.zeros((L,), f32)

        def zr_body(i, c):
            zrow[i, pl.ds(0, L)] = zeros16
            zrow[i, pl.ds(L, L)] = zeros16
            return c
        lax.fori_loop(0, ZR, zr_body, 0)

        def zf_body(j, c):
            pltpu.sync_copy(zrow, ft_sp.at[pl.ds(sid * SLICE + j * ZR, ZR)])
            return c
        lax.fori_loop(0, NZ, zf_body, 0)
        plsc.subcore_barrier()

        def chunk_body(ch, c):
            off = sid * EW3 + ch * CHUNK
            pltpu.sync_copy(src2_hbm.at[pl.ds(cid * EPAD + off, CHUNK)], sidx)
            pltpu.sync_copy(dst_hbm.at[pl.ds(off, CHUNK)], didx)
            pltpu.sync_copy(aexp_hbm.at[pl.ds(off, CHUNK)], aexp_v)
            c1 = pltpu.async_copy(dinv16_hbm.at[didx], dv16, sem)
            c2 = pltpu.async_copy(xsplit_hbm.at[sidx], xsb, sem)
            c1.wait(); c2.wait()

            def e_body(e, cc):
                al = _splat(aexp_v, e) * dv16[e, pl.ds(0, L)]
                stage[e, pl.ds(0, L)] = xsb[e, pl.ds(0, L)] * al
                stage[e, pl.ds(L, L)] = xsb[e, pl.ds(L, L)] * al
                return cc
            lax.fori_loop(0, CHUNK, e_body, 0)

            pltpu.sync_copy(stage, ft_sp.at[didx], add=True)
            return c
        lax.fori_loop(0, NCHUNK3, chunk_body, 0)

        plsc.subcore_barrier()
        pltpu.sync_copy(ft_sp.at[pl.ds(sid * SLICE, SLICE)],
                        ft_hbm.at[pl.ds(cid * NPAD + sid * SLICE, SLICE)])

    return pl.kernel(
        body,
        out_type=jax.ShapeDtypeStruct((NC * NPAD, CH), f32),
        mesh=mesh,
        compiler_params=pltpu.CompilerParams(needs_layout_passes=False, use_tc_tiling_on_sc=False),
        scratch_types=[
            pltpu.VMEM((CHUNK,), i32),
            pltpu.VMEM((CHUNK,), i32),
            pltpu.VMEM((CHUNK,), f32),
            pltpu.VMEM((CHUNK, L), f32),
            pltpu.VMEM((CHUNK, CH), f32),
            pltpu.VMEM((CHUNK, CH), f32),
            pltpu.VMEM((ZR, CH), f32),
            pltpu.VMEM_SHARED((NPAD, CH), f32),
            pltpu.SemaphoreType.DMA,
        ],
    )


def kernel(xij, r, attn, edge_index):
    N, C = xij.shape
    E = edge_index.shape[1]
    NPAD = ((N + 1 + 255) // 256) * 256
    BLK = NC * NS * CHUNK
    EPAD = ((E + BLK - 1) // BLK) * BLK

    src = jnp.concatenate([edge_index[0], jnp.full((EPAD - E,), N, i32)])
    dst = jnp.concatenate([edge_index[1], jnp.full((EPAD - E,), N, i32)])
    xijf = jnp.concatenate([xij, jnp.zeros((NPAD - N, C), f32)], axis=0)
    rT = jnp.concatenate([r.T, jnp.zeros((3, NPAD - N), f32)], axis=1)

    rn = _rn_tc(rT)
    rn16 = jnp.concatenate([rn.T, jnp.zeros((NPAD, L - 3), f32)], axis=1)

    aexp, den = _build_k2(NPAD, EPAD, C)(src, dst, xijf, rn16,
                                         attn.reshape(-1).astype(f32))

    d0 = den[:NPAD].reshape(-1, 128)
    d1 = den[NPAD:].reshape(-1, 128)
    dinv = _dinv_tc(d0, d1).reshape(-1)
    dinv16 = jnp.broadcast_to(dinv[:, None], (NPAD, L))

    CH = C // 2
    xsplit = jnp.concatenate([xijf[:, :CH], xijf[:, CH:]], axis=0)
    src2 = jnp.concatenate([src, src + NPAD])

    ftflat = _build_k3(NPAD, EPAD, CH)(src2, dst, aexp, dinv16, xsplit)
    return jnp.concatenate([ftflat[:N], ftflat[NPAD:NPAD + N]], axis=1)


# speedup vs baseline: 3.7240x; 1.0662x over previous
import functools

import jax
import jax.numpy as jnp
from jax import lax
from jax.experimental import pallas as pl
from jax.experimental.pallas import tpu as pltpu
from jax.experimental.pallas import tpu_sc as plsc

NC = 2
NS = 16
L = 16
CHUNK = 128

f32 = jnp.float32
i32 = jnp.int32


def _splat(ref, i):
    return plsc.load_gather(ref, [jnp.full((L,), i, i32)])


def _rn_tc(rT):
    def body(r_ref, o_ref):
        x = r_ref[...]
        n2 = x[0:1, :] * x[0:1, :] + x[1:2, :] * x[1:2, :] + x[2:3, :] * x[2:3, :]
        o_ref[...] = x * lax.rsqrt(n2 + 1e-35)

    return pl.pallas_call(
        body, out_shape=jax.ShapeDtypeStruct(rT.shape, f32))(rT)


def _dinv_tc(d0, d1):
    def body(a_ref, b_ref, o_ref):
        o_ref[...] = 1.0 / (a_ref[...] + b_ref[...] + 1e-9)

    return pl.pallas_call(
        body, out_shape=jax.ShapeDtypeStruct(d0.shape, f32))(d0, d1)


@functools.lru_cache(maxsize=None)
def _build_k2(NPAD, EPAD, C):
    EW = EPAD // (NC * NS)
    NCHUNK = EW // CHUNK
    SLICE = NPAD // NS
    mesh = plsc.VectorSubcoreMesh(core_axis_name="c", subcore_axis_name="s")

    def body(src_hbm, dst_hbm, xij_hbm, rn16_hbm, attn_hbm,
             aexp_hbm, den_hbm,
             sidx0, didx0, xs0, xd0, rs0, rd0,
             sidx1, didx1, xs1, xd1, rs1, rd1,
             xsum, rnS, rnD, aexp_v, attn_v, zbuf, den_sp, sem0, sem1):
        cid = lax.axis_index("c")
        sid = lax.axis_index("s")
        wid = cid * NS + sid
        zeros16 = jnp.zeros((L,), f32)
        iota_s = lax.iota(i32, L) * CHUNK

        pltpu.sync_copy(attn_hbm, attn_v)

        def zbody(i, c):
            zbuf[pl.ds(i * L, L)] = zeros16
            return c
        lax.fori_loop(0, SLICE // L, zbody, 0)
        pltpu.sync_copy(zbuf, den_sp.at[pl.ds(sid * SLICE, SLICE)])
        plsc.subcore_barrier()

        bufs = ((sidx0, didx0, xs0, xd0, rs0, rd0, sem0),
                (sidx1, didx1, xs1, xd1, rs1, rd1, sem1))

        def issue(off, b):
            sidx, didx, xs, xd, rs16, rd16, sem = b
            pltpu.sync_copy(src_hbm.at[pl.ds(off, CHUNK)], sidx)
            pltpu.sync_copy(dst_hbm.at[pl.ds(off, CHUNK)], didx)
            pltpu.async_copy(xij_hbm.at[sidx], xs, sem)
            pltpu.async_copy(xij_hbm.at[didx], xd, sem)
            pltpu.async_copy(rn16_hbm.at[sidx], rs16, sem)
            pltpu.async_copy(rn16_hbm.at[didx], rd16, sem)

        def drain(b):
            sidx, didx, xs, xd, rs16, rd16, sem = b
            pltpu.make_async_copy(xij_hbm.at[sidx], xs, sem).wait()
            pltpu.make_async_copy(xij_hbm.at[didx], xd, sem).wait()
            pltpu.make_async_copy(rn16_hbm.at[sidx], rs16, sem).wait()
            pltpu.make_async_copy(rn16_hbm.at[didx], rd16, sem).wait()

        def compute(off, b):
            sidx, didx, xs, xd, rs16, rd16, sem = b
            def e_body(e, cc):
                for j in range(C // L):
                    v = xs[e, pl.ds(j * L, L)] + xd[e, pl.ds(j * L, L)]
                    plsc.store_scatter(xsum, [iota_s + (j * L * CHUNK + e)], v)
                plsc.store_scatter(rnS, [iota_s + e], rs16[e, pl.ds(0, L)])
                plsc.store_scatter(rnD, [iota_s + e], rd16[e, pl.ds(0, L)])
                return cc
            lax.fori_loop(0, CHUNK, e_body, 0)

            def group(g, cc):
                g16 = g * L
                cos = (rnS[pl.ds(g16, L)] * rnD[pl.ds(g16, L)]
                       + rnS[pl.ds(CHUNK + g16, L)] * rnD[pl.ds(CHUNK + g16, L)]
                       + rnS[pl.ds(2 * CHUNK + g16, L)] * rnD[pl.ds(2 * CHUNK + g16, L)])
                c2x = cos + cos

                def kbody(k, carry):
                    t0, t1, a = carry
                    s = t1 + xsum[pl.ds(k * CHUNK + g16, L)]
                    a = a + _splat(attn_v, k) * (s / (1.0 + jnp.exp(-s)))
                    return (t1, c2x * t1 - t0, a)

                _, _, acc = lax.fori_loop(
                    0, C, kbody, (cos, jnp.ones((L,), f32),
                                  jnp.zeros((L,), f32)))
                aexp_v[pl.ds(g16, L)] = jnp.exp(acc)
                return cc
            lax.fori_loop(0, CHUNK // L, group, 0)

            pltpu.sync_copy(aexp_v, aexp_hbm.at[pl.ds(off, CHUNK)])
            pltpu.sync_copy(aexp_v, den_sp.at[didx], add=True)

        base = wid * EW
        issue(base, bufs[0])

        def pair(i, c):
            offa = base + (2 * i) * CHUNK
            issue(offa + CHUNK, bufs[1])
            drain(bufs[0])
            compute(offa, bufs[0])
            issue(offa + 2 * CHUNK, bufs[0])
            drain(bufs[1])
            compute(offa + CHUNK, bufs[1])
            return c
        lax.fori_loop(0, NCHUNK // 2, pair, 0)
        drain(bufs[0])

        plsc.subcore_barrier()
        pltpu.sync_copy(den_sp.at[pl.ds(sid * SLICE, SLICE)],
                        den_hbm.at[pl.ds(cid * NPAD + sid * SLICE, SLICE)])

    return pl.kernel(
        body,
        out_type=(jax.ShapeDtypeStruct((EPAD,), f32),
                  jax.ShapeDtypeStruct((NC * NPAD,), f32)),
        mesh=mesh,
        compiler_params=pltpu.CompilerParams(needs_layout_passes=False, use_tc_tiling_on_sc=False),
        scratch_types=[
            pltpu.VMEM((CHUNK,), i32),
            pltpu.VMEM((CHUNK,), i32),
            pltpu.VMEM((CHUNK, C), f32),
            pltpu.VMEM((CHUNK, C), f32),
            pltpu.VMEM((CHUNK, L), f32),
            pltpu.VMEM((CHUNK, L), f32),
            pltpu.VMEM((CHUNK,), i32),
            pltpu.VMEM((CHUNK,), i32),
            pltpu.VMEM((CHUNK, C), f32),
            pltpu.VMEM((CHUNK, C), f32),
            pltpu.VMEM((CHUNK, L), f32),
            pltpu.VMEM((CHUNK, L), f32),
            pltpu.VMEM((C * CHUNK,), f32),
            pltpu.VMEM((L * CHUNK,), f32),
            pltpu.VMEM((L * CHUNK,), f32),
            pltpu.VMEM((CHUNK,), f32),
            pltpu.VMEM((C,), f32),
            pltpu.VMEM((NPAD // NS,), f32),
            pltpu.VMEM_SHARED((NPAD,), f32),
            pltpu.SemaphoreType.DMA,
            pltpu.SemaphoreType.DMA,
        ],
    )


@functools.lru_cache(maxsize=None)
def _build_k3(NPAD, EPAD, CH):
    EW3 = EPAD // NS
    NCHUNK3 = EW3 // CHUNK
    SLICE = NPAD // NS
    ZR = 64 if SLICE % 64 == 0 else 8
    NZ = SLICE // ZR
    mesh = plsc.VectorSubcoreMesh(core_axis_name="c", subcore_axis_name="s")

    def body(src2_hbm, dst_hbm, aexp_hbm, dinv16_hbm, xsplit_hbm,
             ft_hbm,
             sidx, didx, aexp_v, dv16, xsb, stage, zrow, ft_sp,
             sem):
        cid = lax.axis_index("c")
        sid = lax.axis_index("s")
        zeros16 = jnp.zeros((L,), f32)

        def zr_body(i, c):
            zrow[i, pl.ds(0, L)] = zeros16
            zrow[i, pl.ds(L, L)] = zeros16
            return c
        lax.fori_loop(0, ZR, zr_body, 0)

        def zf_body(j, c):
            pltpu.sync_copy(zrow, ft_sp.at[pl.ds(sid * SLICE + j * ZR, ZR)])
            return c
        lax.fori_loop(0, NZ, zf_body, 0)
        plsc.subcore_barrier()

        def chunk_body(ch, c):
            off = sid * EW3 + ch * CHUNK
            pltpu.sync_copy(src2_hbm.at[pl.ds(cid * EPAD + off, CHUNK)], sidx)
            pltpu.sync_copy(dst_hbm.at[pl.ds(off, CHUNK)], didx)
            pltpu.sync_copy(aexp_hbm.at[pl.ds(off, CHUNK)], aexp_v)
            c1 = pltpu.async_copy(dinv16_hbm.at[didx], dv16, sem)
            c2 = pltpu.async_copy(xsplit_hbm.at[sidx], xsb, sem)
            c1.wait(); c2.wait()

            def e_body(e, cc):
                al = _splat(aexp_v, e) * dv16[e, pl.ds(0, L)]
                stage[e, pl.ds(0, L)] = xsb[e, pl.ds(0, L)] * al
                stage[e, pl.ds(L, L)] = xsb[e, pl.ds(L, L)] * al
                return cc
            lax.fori_loop(0, CHUNK, e_body, 0)

            pltpu.sync_copy(stage, ft_sp.at[didx], add=True)
            return c
        lax.fori_loop(0, NCHUNK3, chunk_body, 0)

        plsc.subcore_barrier()
        pltpu.sync_copy(ft_sp.at[pl.ds(sid * SLICE, SLICE)],
                        ft_hbm.at[pl.ds(cid * NPAD + sid * SLICE, SLICE)])

    return pl.kernel(
        body,
        out_type=jax.ShapeDtypeStruct((NC * NPAD, CH), f32),
        mesh=mesh,
        compiler_params=pltpu.CompilerParams(needs_layout_passes=False, use_tc_tiling_on_sc=False),
        scratch_types=[
            pltpu.VMEM((CHUNK,), i32),
            pltpu.VMEM((CHUNK,), i32),
            pltpu.VMEM((CHUNK,), f32),
            pltpu.VMEM((CHUNK, L), f32),
            pltpu.VMEM((CHUNK, CH), f32),
            pltpu.VMEM((CHUNK, CH), f32),
            pltpu.VMEM((ZR, CH), f32),
            pltpu.VMEM_SHARED((NPAD, CH), f32),
            pltpu.SemaphoreType.DMA,
        ],
    )


def kernel(xij, r, attn, edge_index):
    N, C = xij.shape
    E = edge_index.shape[1]
    NPAD = ((N + 1 + 255) // 256) * 256
    BLK = NC * NS * CHUNK
    EPAD = ((E + BLK - 1) // BLK) * BLK

    src = jnp.concatenate([edge_index[0], jnp.full((EPAD - E,), N, i32)])
    dst = jnp.concatenate([edge_index[1], jnp.full((EPAD - E,), N, i32)])
    srck2 = jnp.concatenate([src, jnp.full((CHUNK,), N, i32)])
    dstk2 = jnp.concatenate([dst, jnp.full((CHUNK,), N, i32)])
    xijf = jnp.concatenate([xij, jnp.zeros((NPAD - N, C), f32)], axis=0)
    rT = jnp.concatenate([r.T, jnp.zeros((3, NPAD - N), f32)], axis=1)

    rn = _rn_tc(rT)
    rn16 = jnp.concatenate([rn.T, jnp.zeros((NPAD, L - 3), f32)], axis=1)

    aexp, den = _build_k2(NPAD, EPAD, C)(srck2, dstk2, xijf, rn16,
                                         attn.reshape(-1).astype(f32))

    d0 = den[:NPAD].reshape(-1, 128)
    d1 = den[NPAD:].reshape(-1, 128)
    dinv = _dinv_tc(d0, d1).reshape(-1)
    dinv16 = jnp.broadcast_to(dinv[:, None], (NPAD, L))

    CH = C // 2
    xsplit = jnp.concatenate([xijf[:, :CH], xijf[:, CH:]], axis=0)
    src2 = jnp.concatenate([src, src + NPAD])

    ftflat = _build_k3(NPAD, EPAD, CH)(src2, dst, aexp, dinv16, xsplit)
    return jnp.concatenate([ftflat[:N], ftflat[NPAD:NPAD + N]], axis=1)


# speedup vs baseline: 5.5036x; 1.4779x over previous
import functools

import jax
import jax.numpy as jnp
from jax import lax
from jax.experimental import pallas as pl
from jax.experimental.pallas import tpu as pltpu
from jax.experimental.pallas import tpu_sc as plsc

NC = 2
NS = 16
L = 16
CHUNK = 128

f32 = jnp.float32
i32 = jnp.int32


def _splat(ref, i):
    return plsc.load_gather(ref, [jnp.full((L,), i, i32)])


def _rn_tc(rT):
    def body(r_ref, o_ref):
        x = r_ref[...]
        n2 = x[0:1, :] * x[0:1, :] + x[1:2, :] * x[1:2, :] + x[2:3, :] * x[2:3, :]
        o_ref[...] = x * lax.rsqrt(n2 + 1e-35)

    return pl.pallas_call(
        body, out_shape=jax.ShapeDtypeStruct(rT.shape, f32))(rT)


def _dinv_tc(d0, d1):
    def body(a_ref, b_ref, o_ref):
        o_ref[...] = 1.0 / (a_ref[...] + b_ref[...] + 1e-9)

    return pl.pallas_call(
        body, out_shape=jax.ShapeDtypeStruct(d0.shape, f32))(d0, d1)


@functools.lru_cache(maxsize=None)
def _build_k2(NPAD, EPAD, C):
    EW = EPAD // (NC * NS)
    NCHUNK = EW // CHUNK
    SLICE = NPAD // NS
    mesh = plsc.VectorSubcoreMesh(core_axis_name="c", subcore_axis_name="s")

    def body(src_hbm, dst_hbm, xij_hbm, rn16_hbm, attn_hbm,
             aexp_hbm, den_hbm,
             sidx0, didx0, xs0, xd0, rs0, rd0,
             sidx1, didx1, xs1, xd1, rs1, rd1,
             xsum, rnS, rnD, aexp_v, attn_v, zbuf, den_sp, sem0, sem1):
        cid = lax.axis_index("c")
        sid = lax.axis_index("s")
        wid = cid * NS + sid
        zeros16 = jnp.zeros((L,), f32)
        iota_s = lax.iota(i32, L) * CHUNK

        pltpu.sync_copy(attn_hbm, attn_v)

        def zbody(i, c):
            zbuf[pl.ds(i * L, L)] = zeros16
            return c
        lax.fori_loop(0, SLICE // L, zbody, 0)
        pltpu.sync_copy(zbuf, den_sp.at[pl.ds(sid * SLICE, SLICE)])
        plsc.subcore_barrier()

        bufs = ((sidx0, didx0, xs0, xd0, rs0, rd0, sem0),
                (sidx1, didx1, xs1, xd1, rs1, rd1, sem1))

        def issue(off, b):
            sidx, didx, xs, xd, rs16, rd16, sem = b
            pltpu.sync_copy(src_hbm.at[pl.ds(off, CHUNK)], sidx)
            pltpu.sync_copy(dst_hbm.at[pl.ds(off, CHUNK)], didx)
            pltpu.async_copy(xij_hbm.at[sidx], xs, sem)
            pltpu.async_copy(xij_hbm.at[didx], xd, sem)
            pltpu.async_copy(rn16_hbm.at[sidx], rs16, sem)
            pltpu.async_copy(rn16_hbm.at[didx], rd16, sem)

        def drain(b):
            sidx, didx, xs, xd, rs16, rd16, sem = b
            pltpu.make_async_copy(xij_hbm.at[sidx], xs, sem).wait()
            pltpu.make_async_copy(xij_hbm.at[didx], xd, sem).wait()
            pltpu.make_async_copy(rn16_hbm.at[sidx], rs16, sem).wait()
            pltpu.make_async_copy(rn16_hbm.at[didx], rd16, sem).wait()

        def compute(off, b):
            sidx, didx, xs, xd, rs16, rd16, sem = b
            def e_body(e4, cc):
                for d in range(4):
                    e = e4 * 4 + d
                    for j in range(C // L):
                        v = xs[e, pl.ds(j * L, L)] + xd[e, pl.ds(j * L, L)]
                        plsc.store_scatter(
                            xsum, [iota_s + (j * L * CHUNK + e)], v)
                    plsc.store_scatter(rnS, [iota_s + e], rs16[e, pl.ds(0, L)])
                    plsc.store_scatter(rnD, [iota_s + e], rd16[e, pl.ds(0, L)])
                return cc
            lax.fori_loop(0, CHUNK // 4, e_body, 0)

            def group(g, cc):
                g16 = g * L
                cos = (rnS[pl.ds(g16, L)] * rnD[pl.ds(g16, L)]
                       + rnS[pl.ds(CHUNK + g16, L)] * rnD[pl.ds(CHUNK + g16, L)]
                       + rnS[pl.ds(2 * CHUNK + g16, L)] * rnD[pl.ds(2 * CHUNK + g16, L)])
                c2x = cos + cos

                def kbody(k8, carry):
                    t0, t1, a = carry
                    terms = []
                    for d in range(8):
                        k = k8 * 8 + d
                        s = t1 + xsum[pl.ds(k * CHUNK + g16, L)]
                        terms.append(_splat(attn_v, k)
                                     * (s / (1.0 + jnp.exp(-s))))
                        t0, t1 = t1, c2x * t1 - t0
                    p = ((terms[0] + terms[1]) + (terms[2] + terms[3])) + (
                        (terms[4] + terms[5]) + (terms[6] + terms[7]))
                    return (t0, t1, a + p)

                _, _, acc = lax.fori_loop(
                    0, C // 8, kbody, (cos, jnp.ones((L,), f32),
                                       jnp.zeros((L,), f32)))
                aexp_v[pl.ds(g16, L)] = jnp.exp(acc)
                return cc
            lax.fori_loop(0, CHUNK // L, group, 0)

            pltpu.sync_copy(aexp_v, aexp_hbm.at[pl.ds(off, CHUNK)])
            pltpu.sync_copy(aexp_v, den_sp.at[didx], add=True)

        base = wid * EW
        issue(base, bufs[0])

        def pair(i, c):
            offa = base + (2 * i) * CHUNK
            issue(offa + CHUNK, bufs[1])
            drain(bufs[0])
            compute(offa, bufs[0])
            issue(offa + 2 * CHUNK, bufs[0])
            drain(bufs[1])
            compute(offa + CHUNK, bufs[1])
            return c
        lax.fori_loop(0, NCHUNK // 2, pair, 0)
        drain(bufs[0])

        plsc.subcore_barrier()
        pltpu.sync_copy(den_sp.at[pl.ds(sid * SLICE, SLICE)],
                        den_hbm.at[pl.ds(cid * NPAD + sid * SLICE, SLICE)])

    return pl.kernel(
        body,
        out_type=(jax.ShapeDtypeStruct((EPAD,), f32),
                  jax.ShapeDtypeStruct((NC * NPAD,), f32)),
        mesh=mesh,
        compiler_params=pltpu.CompilerParams(needs_layout_passes=False, use_tc_tiling_on_sc=False),
        scratch_types=[
            pltpu.VMEM((CHUNK,), i32),
            pltpu.VMEM((CHUNK,), i32),
            pltpu.VMEM((CHUNK, C), f32),
            pltpu.VMEM((CHUNK, C), f32),
            pltpu.VMEM((CHUNK, L), f32),
            pltpu.VMEM((CHUNK, L), f32),
            pltpu.VMEM((CHUNK,), i32),
            pltpu.VMEM((CHUNK,), i32),
            pltpu.VMEM((CHUNK, C), f32),
            pltpu.VMEM((CHUNK, C), f32),
            pltpu.VMEM((CHUNK, L), f32),
            pltpu.VMEM((CHUNK, L), f32),
            pltpu.VMEM((C * CHUNK,), f32),
            pltpu.VMEM((L * CHUNK,), f32),
            pltpu.VMEM((L * CHUNK,), f32),
            pltpu.VMEM((CHUNK,), f32),
            pltpu.VMEM((C,), f32),
            pltpu.VMEM((NPAD // NS,), f32),
            pltpu.VMEM_SHARED((NPAD,), f32),
            pltpu.SemaphoreType.DMA,
            pltpu.SemaphoreType.DMA,
        ],
    )


@functools.lru_cache(maxsize=None)
def _build_k3(NPAD, EPAD, CH):
    EW3 = EPAD // NS
    NCHUNK3 = EW3 // CHUNK
    SLICE = NPAD // NS
    ZR = 64 if SLICE % 64 == 0 else 8
    NZ = SLICE // ZR
    mesh = plsc.VectorSubcoreMesh(core_axis_name="c", subcore_axis_name="s")

    def body(src2_hbm, dst_hbm, aexp_hbm, dinv16_hbm, xsplit_hbm,
             ft_hbm,
             sidx, didx, aexp_v, dv16, xsb, stage, zrow, ft_sp,
             sem):
        cid = lax.axis_index("c")
        sid = lax.axis_index("s")
        zeros16 = jnp.zeros((L,), f32)

        def zr_body(i, c):
            zrow[i, pl.ds(0, L)] = zeros16
            zrow[i, pl.ds(L, L)] = zeros16
            return c
        lax.fori_loop(0, ZR, zr_body, 0)

        def zf_body(j, c):
            pltpu.sync_copy(zrow, ft_sp.at[pl.ds(sid * SLICE + j * ZR, ZR)])
            return c
        lax.fori_loop(0, NZ, zf_body, 0)
        plsc.subcore_barrier()

        def chunk_body(ch, c):
            off = sid * EW3 + ch * CHUNK
            pltpu.sync_copy(src2_hbm.at[pl.ds(cid * EPAD + off, CHUNK)], sidx)
            pltpu.sync_copy(dst_hbm.at[pl.ds(off, CHUNK)], didx)
            pltpu.sync_copy(aexp_hbm.at[pl.ds(off, CHUNK)], aexp_v)
            c1 = pltpu.async_copy(dinv16_hbm.at[didx], dv16, sem)
            c2 = pltpu.async_copy(xsplit_hbm.at[sidx], xsb, sem)
            c1.wait(); c2.wait()

            def e_body(e4, cc):
                for d in range(4):
                    e = e4 * 4 + d
                    al = _splat(aexp_v, e) * dv16[e, pl.ds(0, L)]
                    stage[e, pl.ds(0, L)] = xsb[e, pl.ds(0, L)] * al
                    stage[e, pl.ds(L, L)] = xsb[e, pl.ds(L, L)] * al
                return cc
            lax.fori_loop(0, CHUNK // 4, e_body, 0)

            pltpu.sync_copy(stage, ft_sp.at[didx], add=True)
            return c
        lax.fori_loop(0, NCHUNK3, chunk_body, 0)

        plsc.subcore_barrier()
        pltpu.sync_copy(ft_sp.at[pl.ds(sid * SLICE, SLICE)],
                        ft_hbm.at[pl.ds(cid * NPAD + sid * SLICE, SLICE)])

    return pl.kernel(
        body,
        out_type=jax.ShapeDtypeStruct((NC * NPAD, CH), f32),
        mesh=mesh,
        compiler_params=pltpu.CompilerParams(needs_layout_passes=False, use_tc_tiling_on_sc=False),
        scratch_types=[
            pltpu.VMEM((CHUNK,), i32),
            pltpu.VMEM((CHUNK,), i32),
            pltpu.VMEM((CHUNK,), f32),
            pltpu.VMEM((CHUNK, L), f32),
            pltpu.VMEM((CHUNK, CH), f32),
            pltpu.VMEM((CHUNK, CH), f32),
            pltpu.VMEM((ZR, CH), f32),
            pltpu.VMEM_SHARED((NPAD, CH), f32),
            pltpu.SemaphoreType.DMA,
        ],
    )


def kernel(xij, r, attn, edge_index):
    N, C = xij.shape
    E = edge_index.shape[1]
    NPAD = ((N + 1 + 255) // 256) * 256
    BLK = NC * NS * CHUNK
    EPAD = ((E + BLK - 1) // BLK) * BLK

    src = jnp.concatenate([edge_index[0], jnp.full((EPAD - E,), N, i32)])
    dst = jnp.concatenate([edge_index[1], jnp.full((EPAD - E,), N, i32)])
    srck2 = jnp.concatenate([src, jnp.full((CHUNK,), N, i32)])
    dstk2 = jnp.concatenate([dst, jnp.full((CHUNK,), N, i32)])
    xijf = jnp.concatenate([xij, jnp.zeros((NPAD - N, C), f32)], axis=0)
    rT = jnp.concatenate([r.T, jnp.zeros((3, NPAD - N), f32)], axis=1)

    rn = _rn_tc(rT)
    rn16 = jnp.concatenate([rn.T, jnp.zeros((NPAD, L - 3), f32)], axis=1)

    aexp, den = _build_k2(NPAD, EPAD, C)(srck2, dstk2, xijf, rn16,
                                         attn.reshape(-1).astype(f32))

    d0 = den[:NPAD].reshape(-1, 128)
    d1 = den[NPAD:].reshape(-1, 128)
    dinv = _dinv_tc(d0, d1).reshape(-1)
    dinv16 = jnp.broadcast_to(dinv[:, None], (NPAD, L))

    CH = C // 2
    xsplit = jnp.concatenate([xijf[:, :CH], xijf[:, CH:]], axis=0)
    src2 = jnp.concatenate([src, src + NPAD])

    ftflat = _build_k3(NPAD, EPAD, CH)(src2, dst, aexp, dinv16, xsplit)
    return jnp.concatenate([ftflat[:N], ftflat[NPAD:NPAD + N]], axis=1)


# speedup vs baseline: 5.5115x; 1.0014x over previous
import functools

import jax
import jax.numpy as jnp
from jax import lax
from jax.experimental import pallas as pl
from jax.experimental.pallas import tpu as pltpu
from jax.experimental.pallas import tpu_sc as plsc

NC = 2
NS = 16
L = 16
CHUNK = 128

f32 = jnp.float32
i32 = jnp.int32


def _splat(ref, i):
    return plsc.load_gather(ref, [jnp.full((L,), i, i32)])


def _rn_tc(rT):
    def body(r_ref, o_ref):
        x = r_ref[...]
        n2 = x[0:1, :] * x[0:1, :] + x[1:2, :] * x[1:2, :] + x[2:3, :] * x[2:3, :]
        o_ref[...] = x * lax.rsqrt(n2 + 1e-35)

    return pl.pallas_call(
        body, out_shape=jax.ShapeDtypeStruct(rT.shape, f32))(rT)


def _dinv_tc(d0, d1):
    def body(a_ref, b_ref, o_ref):
        o_ref[...] = 1.0 / (a_ref[...] + b_ref[...] + 1e-9)

    return pl.pallas_call(
        body, out_shape=jax.ShapeDtypeStruct(d0.shape, f32))(d0, d1)


@functools.lru_cache(maxsize=None)
def _build_k2(NPAD, EPAD, C):
    EW = EPAD // (NC * NS)
    NCHUNK = EW // CHUNK
    SLICE = NPAD // NS
    mesh = plsc.VectorSubcoreMesh(core_axis_name="c", subcore_axis_name="s")

    def body(src_hbm, dst_hbm, xij_hbm, rn16_hbm, attn_hbm,
             aexp_hbm, den_hbm,
             sidx0, didx0, xs0, xd0, rs0, rd0,
             sidx1, didx1, xs1, xd1, rs1, rd1,
             xsum, rnS, rnD, aexp_v, attn_v, zbuf, den_sp, sem0, sem1):
        cid = lax.axis_index("c")
        sid = lax.axis_index("s")
        wid = cid * NS + sid
        zeros16 = jnp.zeros((L,), f32)
        iota_s = lax.iota(i32, L) * CHUNK

        pltpu.sync_copy(attn_hbm, attn_v)

        def zbody(i, c):
            zbuf[pl.ds(i * L, L)] = zeros16
            return c
        lax.fori_loop(0, SLICE // L, zbody, 0)
        pltpu.sync_copy(zbuf, den_sp.at[pl.ds(sid * SLICE, SLICE)])
        plsc.subcore_barrier()

        bufs = ((sidx0, didx0, xs0, xd0, rs0, rd0, sem0),
                (sidx1, didx1, xs1, xd1, rs1, rd1, sem1))

        def issue(off, b):
            sidx, didx, xs, xd, rs16, rd16, sem = b
            pltpu.sync_copy(src_hbm.at[pl.ds(off, CHUNK)], sidx)
            pltpu.sync_copy(dst_hbm.at[pl.ds(off, CHUNK)], didx)
            pltpu.async_copy(xij_hbm.at[sidx], xs, sem)
            pltpu.async_copy(xij_hbm.at[didx], xd, sem)
            pltpu.async_copy(rn16_hbm.at[sidx], rs16, sem)
            pltpu.async_copy(rn16_hbm.at[didx], rd16, sem)

        def drain(b):
            sidx, didx, xs, xd, rs16, rd16, sem = b
            pltpu.make_async_copy(xij_hbm.at[sidx], xs, sem).wait()
            pltpu.make_async_copy(xij_hbm.at[didx], xd, sem).wait()
            pltpu.make_async_copy(rn16_hbm.at[sidx], rs16, sem).wait()
            pltpu.make_async_copy(rn16_hbm.at[didx], rd16, sem).wait()

        def compute(off, b):
            sidx, didx, xs, xd, rs16, rd16, sem = b
            def e_body(e4, cc):
                for d in range(8):
                    e = e4 * 8 + d
                    for j in range(C // L):
                        v = xs[e, pl.ds(j * L, L)] + xd[e, pl.ds(j * L, L)]
                        plsc.store_scatter(
                            xsum, [iota_s + (j * L * CHUNK + e)], v)
                    plsc.store_scatter(rnS, [iota_s + e], rs16[e, pl.ds(0, L)])
                    plsc.store_scatter(rnD, [iota_s + e], rd16[e, pl.ds(0, L)])
                return cc
            lax.fori_loop(0, CHUNK // 8, e_body, 0)

            def group(g, cc):
                g16 = g * L
                cos = (rnS[pl.ds(g16, L)] * rnD[pl.ds(g16, L)]
                       + rnS[pl.ds(CHUNK + g16, L)] * rnD[pl.ds(CHUNK + g16, L)]
                       + rnS[pl.ds(2 * CHUNK + g16, L)] * rnD[pl.ds(2 * CHUNK + g16, L)])
                c2x = cos + cos

                def kbody(k8, carry):
                    t0, t1, a = carry
                    terms = []
                    for d in range(16):
                        k = k8 * 16 + d
                        s = t1 + xsum[pl.ds(k * CHUNK + g16, L)]
                        terms.append(_splat(attn_v, k)
                                     * (s / (1.0 + jnp.exp(-s))))
                        t0, t1 = t1, c2x * t1 - t0
                    while len(terms) > 1:
                        terms = [terms[i] + terms[i + 1]
                                 for i in range(0, len(terms), 2)]
                    return (t0, t1, a + terms[0])

                _, _, acc = lax.fori_loop(
                    0, C // 16, kbody, (cos, jnp.ones((L,), f32),
                                       jnp.zeros((L,), f32)))
                aexp_v[pl.ds(g16, L)] = jnp.exp(acc)
                return cc
            lax.fori_loop(0, CHUNK // L, group, 0)

            pltpu.sync_copy(aexp_v, aexp_hbm.at[pl.ds(off, CHUNK)])
            pltpu.sync_copy(aexp_v, den_sp.at[didx], add=True)

        base = wid * EW
        issue(base, bufs[0])

        def pair(i, c):
            offa = base + (2 * i) * CHUNK
            issue(offa + CHUNK, bufs[1])
            drain(bufs[0])
            compute(offa, bufs[0])
            issue(offa + 2 * CHUNK, bufs[0])
            drain(bufs[1])
            compute(offa + CHUNK, bufs[1])
            return c
        lax.fori_loop(0, NCHUNK // 2, pair, 0)
        drain(bufs[0])

        plsc.subcore_barrier()
        pltpu.sync_copy(den_sp.at[pl.ds(sid * SLICE, SLICE)],
                        den_hbm.at[pl.ds(cid * NPAD + sid * SLICE, SLICE)])

    return pl.kernel(
        body,
        out_type=(jax.ShapeDtypeStruct((EPAD,), f32),
                  jax.ShapeDtypeStruct((NC * NPAD,), f32)),
        mesh=mesh,
        compiler_params=pltpu.CompilerParams(needs_layout_passes=False, use_tc_tiling_on_sc=False),
        scratch_types=[
            pltpu.VMEM((CHUNK,), i32),
            pltpu.VMEM((CHUNK,), i32),
            pltpu.VMEM((CHUNK, C), f32),
            pltpu.VMEM((CHUNK, C), f32),
            pltpu.VMEM((CHUNK, L), f32),
            pltpu.VMEM((CHUNK, L), f32),
            pltpu.VMEM((CHUNK,), i32),
            pltpu.VMEM((CHUNK,), i32),
            pltpu.VMEM((CHUNK, C), f32),
            pltpu.VMEM((CHUNK, C), f32),
            pltpu.VMEM((CHUNK, L), f32),
            pltpu.VMEM((CHUNK, L), f32),
            pltpu.VMEM((C * CHUNK,), f32),
            pltpu.VMEM((L * CHUNK,), f32),
            pltpu.VMEM((L * CHUNK,), f32),
            pltpu.VMEM((CHUNK,), f32),
            pltpu.VMEM((C,), f32),
            pltpu.VMEM((NPAD // NS,), f32),
            pltpu.VMEM_SHARED((NPAD,), f32),
            pltpu.SemaphoreType.DMA,
            pltpu.SemaphoreType.DMA,
        ],
    )


@functools.lru_cache(maxsize=None)
def _build_k3(NPAD, EPAD, CH):
    EW3 = EPAD // NS
    NCHUNK3 = EW3 // CHUNK
    SLICE = NPAD // NS
    ZR = 64 if SLICE % 64 == 0 else 8
    NZ = SLICE // ZR
    mesh = plsc.VectorSubcoreMesh(core_axis_name="c", subcore_axis_name="s")

    def body(src2_hbm, dst_hbm, aexp_hbm, dinv16_hbm, xsplit_hbm,
             ft_hbm,
             sidx, didx, aexp_v, dv16, xsb, stage, zrow, ft_sp,
             sem):
        cid = lax.axis_index("c")
        sid = lax.axis_index("s")
        zeros16 = jnp.zeros((L,), f32)

        def zr_body(i, c):
            zrow[i, pl.ds(0, L)] = zeros16
            zrow[i, pl.ds(L, L)] = zeros16
            return c
        lax.fori_loop(0, ZR, zr_body, 0)

        def zf_body(j, c):
            pltpu.sync_copy(zrow, ft_sp.at[pl.ds(sid * SLICE + j * ZR, ZR)])
            return c
        lax.fori_loop(0, NZ, zf_body, 0)
        plsc.subcore_barrier()

        def chunk_body(ch, c):
            off = sid * EW3 + ch * CHUNK
            pltpu.sync_copy(src2_hbm.at[pl.ds(cid * EPAD + off, CHUNK)], sidx)
            pltpu.sync_copy(dst_hbm.at[pl.ds(off, CHUNK)], didx)
            pltpu.sync_copy(aexp_hbm.at[pl.ds(off, CHUNK)], aexp_v)
            c1 = pltpu.async_copy(dinv16_hbm.at[didx], dv16, sem)
            c2 = pltpu.async_copy(xsplit_hbm.at[sidx], xsb, sem)
            c1.wait(); c2.wait()

            def e_body(e4, cc):
                for d in range(4):
                    e = e4 * 4 + d
                    al = _splat(aexp_v, e) * dv16[e, pl.ds(0, L)]
                    stage[e, pl.ds(0, L)] = xsb[e, pl.ds(0, L)] * al
                    stage[e, pl.ds(L, L)] = xsb[e, pl.ds(L, L)] * al
                return cc
            lax.fori_loop(0, CHUNK // 4, e_body, 0)

            pltpu.sync_copy(stage, ft_sp.at[didx], add=True)
            return c
        lax.fori_loop(0, NCHUNK3, chunk_body, 0)

        plsc.subcore_barrier()
        pltpu.sync_copy(ft_sp.at[pl.ds(sid * SLICE, SLICE)],
                        ft_hbm.at[pl.ds(cid * NPAD + sid * SLICE, SLICE)])

    return pl.kernel(
        body,
        out_type=jax.ShapeDtypeStruct((NC * NPAD, CH), f32),
        mesh=mesh,
        compiler_params=pltpu.CompilerParams(needs_layout_passes=False, use_tc_tiling_on_sc=False),
        scratch_types=[
            pltpu.VMEM((CHUNK,), i32),
            pltpu.VMEM((CHUNK,), i32),
            pltpu.VMEM((CHUNK,), f32),
            pltpu.VMEM((CHUNK, L), f32),
            pltpu.VMEM((CHUNK, CH), f32),
            pltpu.VMEM((CHUNK, CH), f32),
            pltpu.VMEM((ZR, CH), f32),
            pltpu.VMEM_SHARED((NPAD, CH), f32),
            pltpu.SemaphoreType.DMA,
        ],
    )


def kernel(xij, r, attn, edge_index):
    N, C = xij.shape
    E = edge_index.shape[1]
    NPAD = ((N + 1 + 255) // 256) * 256
    BLK = NC * NS * CHUNK
    EPAD = ((E + BLK - 1) // BLK) * BLK

    src = jnp.concatenate([edge_index[0], jnp.full((EPAD - E,), N, i32)])
    dst = jnp.concatenate([edge_index[1], jnp.full((EPAD - E,), N, i32)])
    srck2 = jnp.concatenate([src, jnp.full((CHUNK,), N, i32)])
    dstk2 = jnp.concatenate([dst, jnp.full((CHUNK,), N, i32)])
    xijf = jnp.concatenate([xij, jnp.zeros((NPAD - N, C), f32)], axis=0)
    rT = jnp.concatenate([r.T, jnp.zeros((3, NPAD - N), f32)], axis=1)

    rn = _rn_tc(rT)
    rn16 = jnp.concatenate([rn.T, jnp.zeros((NPAD, L - 3), f32)], axis=1)

    aexp, den = _build_k2(NPAD, EPAD, C)(srck2, dstk2, xijf, rn16,
                                         attn.reshape(-1).astype(f32))

    d0 = den[:NPAD].reshape(-1, 128)
    d1 = den[NPAD:].reshape(-1, 128)
    dinv = _dinv_tc(d0, d1).reshape(-1)
    dinv16 = jnp.broadcast_to(dinv[:, None], (NPAD, L))

    CH = C // 2
    xsplit = jnp.concatenate([xijf[:, :CH], xijf[:, CH:]], axis=0)
    src2 = jnp.concatenate([src, src + NPAD])

    ftflat = _build_k3(NPAD, EPAD, CH)(src2, dst, aexp, dinv16, xsplit)
    return jnp.concatenate([ftflat[:N], ftflat[NPAD:NPAD + N]], axis=1)
